# Initial kernel scaffold; baseline (speedup 1.0000x reference)
import jax
import jax.numpy as jnp
from jax.experimental import pallas as pl


def kernel(x, atom_to_cycle, emb_weight):
    raise NotImplementedError("write your pallas kernel here")


# speedup vs baseline: 29.0864x; 29.0864x over previous
import functools

import jax
import jax.numpy as jnp
from jax import lax
from jax.experimental import pallas as pl
from jax.experimental.pallas import tpu as pltpu
from jax.experimental.pallas import tpu_sc as plsc

N_ATOMS = 10000
N_EDGES = 320000
HIDDEN = 128
VOCAB = 28
NUM_SEGMENTS = 10000

VOCAB_PAD = 32
NC = 2
NS = 16
NW = NC * NS
E_PER_W = N_EDGES // NW
CHUNK = 128
N_CHUNKS = -(-E_PER_W // CHUNK)
HIST_FLAT = NUM_SEGMENTS * VOCAB_PAD
PAD_SLOT = HIST_FLAT
HIST_SP = 320256
ZSLICE = HIST_SP // NS
OSLICE = HIST_FLAT // NS


def _sc_body(x_hbm, a2c_hbm, hist_hbm,
             x_v, src_v, dst_v, fidx_v, ones_v, zer_v, hist_s):
    c = lax.axis_index("c")
    s = lax.axis_index("s")
    w = c * NS + s
    base = w * E_PER_W

    zero16 = jnp.zeros((16,), jnp.float32)

    def zbody(i, carry):
        zer_v[pl.ds(i * 16, 16)] = zero16
        return carry

    lax.fori_loop(0, ZSLICE // 16, zbody, 0)
    pltpu.sync_copy(zer_v, hist_s.at[pl.ds(s * ZSLICE, ZSLICE)])

    one16 = jnp.ones((16,), jnp.float32)
    for kk in range(CHUNK // 16):
        ones_v[pl.ds(kk * 16, 16)] = one16

    pltpu.sync_copy(x_hbm, x_v)
    pltpu.sync_copy(a2c_hbm.at[pl.ds(base, E_PER_W)], src_v)
    pltpu.sync_copy(a2c_hbm.at[pl.ds(N_EDGES + base, E_PER_W)], dst_v)

    def cbody(j, carry):
        s16 = src_v[pl.ds(j * 16, 16)]
        d16 = dst_v[pl.ds(j * 16, 16)]
        t16 = plsc.load_gather(x_v, [s16])
        f16 = d16 * VOCAB_PAD + t16
        fidx_v[j // 8, pl.ds((j % 8) * 16, 16)] = f16
        return carry

    lax.fori_loop(0, E_PER_W // 16, cbody, 0)

    padv = jnp.full((16,), PAD_SLOT, jnp.int32)
    for kk in range(1, CHUNK // 16):
        fidx_v[N_CHUNKS - 1, pl.ds(kk * 16, 16)] = padv

    plsc.subcore_barrier()

    for j in range(N_CHUNKS):
        pltpu.sync_copy(ones_v, hist_s.at[fidx_v.at[j]], add=True)

    plsc.subcore_barrier()

    pltpu.sync_copy(hist_s.at[pl.ds(s * OSLICE, OSLICE)],
                    zer_v.at[pl.ds(0, OSLICE)])
    pltpu.sync_copy(zer_v.at[pl.ds(0, OSLICE)],
                    hist_hbm.at[pl.ds(c * HIST_FLAT + s * OSLICE, OSLICE)])


_sc_hist = pl.kernel(
    _sc_body,
    out_type=jax.ShapeDtypeStruct((NC * HIST_FLAT,), jnp.float32),
    mesh=plsc.VectorSubcoreMesh(core_axis_name="c", subcore_axis_name="s"),
    compiler_params=pltpu.CompilerParams(needs_layout_passes=False),
    scratch_types=[
        pltpu.VMEM((N_ATOMS,), jnp.int32),
        pltpu.VMEM((E_PER_W,), jnp.int32),
        pltpu.VMEM((E_PER_W,), jnp.int32),
        pltpu.VMEM((N_CHUNKS, CHUNK), jnp.int32),
        pltpu.VMEM((CHUNK,), jnp.float32),
        pltpu.VMEM((ZSLICE,), jnp.float32),
        pltpu.VMEM_SHARED((HIST_SP,), jnp.float32),
    ],
)


MM_BLK = 2000


def _mm_body(hist_ref, emb_ref, out_ref):
    h = hist_ref[0] + hist_ref[1]
    out_ref[...] = jnp.dot(h, emb_ref[...], preferred_element_type=jnp.float32)


def _tc_expand(hist3, embp):
    return pl.pallas_call(
        _mm_body,
        grid=(NUM_SEGMENTS // MM_BLK,),
        in_specs=[
            pl.BlockSpec((NC, MM_BLK, VOCAB_PAD), lambda i: (0, i, 0)),
            pl.BlockSpec((VOCAB_PAD, HIDDEN), lambda i: (0, 0)),
        ],
        out_specs=pl.BlockSpec((MM_BLK, HIDDEN), lambda i: (i, 0)),
        out_shape=jax.ShapeDtypeStruct((NUM_SEGMENTS, HIDDEN), jnp.float32),
    )(hist3, embp)


@jax.jit
def kernel(x, atom_to_cycle, emb_weight):
    hist = _sc_hist(x, atom_to_cycle.reshape(2 * N_EDGES))
    hist3 = hist.reshape(NC, NUM_SEGMENTS, VOCAB_PAD)
    embp = jnp.pad(emb_weight, ((0, VOCAB_PAD - VOCAB), (0, 0)))
    return _tc_expand(hist3, embp)


# speedup vs baseline: 40.5528x; 1.3942x over previous
import functools

import jax
import jax.numpy as jnp
from jax import lax
from jax.experimental import pallas as pl
from jax.experimental.pallas import tpu as pltpu
from jax.experimental.pallas import tpu_sc as plsc

N_ATOMS = 10000
N_EDGES = 320000
HIDDEN = 128
VOCAB = 28
NUM_SEGMENTS = 10000

VOCAB_PAD = 32
NC = 2
NS = 16
NW = NC * NS
E_PER_W = N_EDGES // NW
CHUNK = 128
N_CHUNKS = -(-E_PER_W // CHUNK)
E_PAD = N_CHUNKS * CHUNK
HIST_FLAT = NUM_SEGMENTS * VOCAB_PAD
HIST_SP = 321536
ZSLICE = HIST_SP // NS
OSLICE = HIST_FLAT // NS


def _sc_body(x_hbm, a2c_hbm, hist_hbm,
             x_v, src_v, dst_v, fidx_v, ones_v, zer_v, hist_s,
             sem_in, sem_sc):
    c = lax.axis_index("c")
    s = lax.axis_index("s")
    w = c * NS + s
    base = w * E_PER_W

    cp_x = pltpu.async_copy(x_hbm, x_v, sem_in)
    cp_s = pltpu.async_copy(a2c_hbm.at[pl.ds(base, E_PER_W)],
                            src_v.at[pl.ds(0, E_PER_W)], sem_in)
    cp_d = pltpu.async_copy(a2c_hbm.at[pl.ds(N_EDGES + base, E_PER_W)],
                            dst_v.at[pl.ds(0, E_PER_W)], sem_in)

    zero16 = jnp.zeros((16,), jnp.float32)

    def zbody(i, carry):
        for k in range(8):
            zer_v[pl.ds(i * 128 + k * 16, 16)] = zero16
        return carry

    lax.fori_loop(0, ZSLICE // 128, zbody, 0)
    pltpu.sync_copy(zer_v, hist_s.at[pl.ds(s * ZSLICE, ZSLICE)])

    one16 = jnp.ones((16,), jnp.float32)
    for k in range(CHUNK // 16):
        ones_v[pl.ds(k * 16, 16)] = one16

    cp_x.wait()
    cp_s.wait()
    cp_d.wait()

    pad_src = jnp.zeros((16,), jnp.int32)
    pad_dst = jnp.full((16,), NUM_SEGMENTS, jnp.int32)
    for k in range((E_PAD - E_PER_W) // 16):
        src_v[pl.ds(E_PER_W + k * 16, 16)] = pad_src
        dst_v[pl.ds(E_PER_W + k * 16, 16)] = pad_dst

    def cbody(j, carry):
        for k in range(CHUNK // 16):
            s16 = src_v[pl.ds(j * CHUNK + k * 16, 16)]
            d16 = dst_v[pl.ds(j * CHUNK + k * 16, 16)]
            t16 = plsc.load_gather(x_v, [s16])
            fidx_v[j, pl.ds(k * 16, 16)] = d16 * VOCAB_PAD + t16
        pltpu.async_copy(ones_v, hist_s.at[fidx_v.at[j]], sem_sc, add=True)
        return carry

    lax.fori_loop(0, N_CHUNKS, cbody, 0)

    def dbody(j, carry):
        pltpu.make_async_copy(ones_v, hist_s.at[fidx_v.at[0]], sem_sc).wait()
        return carry

    lax.fori_loop(0, N_CHUNKS, dbody, 0)

    plsc.subcore_barrier()

    pltpu.sync_copy(hist_s.at[pl.ds(s * OSLICE, OSLICE)],
                    zer_v.at[pl.ds(0, OSLICE)])
    pltpu.sync_copy(zer_v.at[pl.ds(0, OSLICE)],
                    hist_hbm.at[pl.ds(c * HIST_FLAT + s * OSLICE, OSLICE)])


_sc_hist = pl.kernel(
    _sc_body,
    out_type=jax.ShapeDtypeStruct((NC * HIST_FLAT,), jnp.float32),
    mesh=plsc.VectorSubcoreMesh(core_axis_name="c", subcore_axis_name="s"),
    compiler_params=pltpu.CompilerParams(needs_layout_passes=False),
    scratch_types=[
        pltpu.VMEM((N_ATOMS,), jnp.int32),
        pltpu.VMEM((E_PAD,), jnp.int32),
        pltpu.VMEM((E_PAD,), jnp.int32),
        pltpu.VMEM((N_CHUNKS, CHUNK), jnp.int32),
        pltpu.VMEM((CHUNK,), jnp.float32),
        pltpu.VMEM((ZSLICE,), jnp.float32),
        pltpu.VMEM_SHARED((HIST_SP,), jnp.float32),
        pltpu.SemaphoreType.DMA,
        pltpu.SemaphoreType.DMA,
    ],
)


MM_BLK = 2000


def _mm_body(hist_ref, emb_ref, out_ref):
    h = hist_ref[0] + hist_ref[1]
    out_ref[...] = jnp.dot(h, emb_ref[...], preferred_element_type=jnp.float32)


def _tc_expand(hist3, embp):
    return pl.pallas_call(
        _mm_body,
        grid=(NUM_SEGMENTS // MM_BLK,),
        in_specs=[
            pl.BlockSpec((NC, MM_BLK, VOCAB_PAD), lambda i: (0, i, 0)),
            pl.BlockSpec((VOCAB_PAD, HIDDEN), lambda i: (0, 0)),
        ],
        out_specs=pl.BlockSpec((MM_BLK, HIDDEN), lambda i: (i, 0)),
        out_shape=jax.ShapeDtypeStruct((NUM_SEGMENTS, HIDDEN), jnp.float32),
    )(hist3, embp)


@jax.jit
def kernel(x, atom_to_cycle, emb_weight):
    hist = _sc_hist(x, atom_to_cycle.reshape(2 * N_EDGES))
    hist3 = hist.reshape(NC, NUM_SEGMENTS, VOCAB_PAD)
    embp = jnp.pad(emb_weight, ((0, VOCAB_PAD - VOCAB), (0, 0)))
    return _tc_expand(hist3, embp)


# speedup vs baseline: 46.8030x; 1.1541x over previous
import functools

import jax
import jax.numpy as jnp
from jax import lax
from jax.experimental import pallas as pl
from jax.experimental.pallas import tpu as pltpu
from jax.experimental.pallas import tpu_sc as plsc

N_ATOMS = 10000
N_EDGES = 320000
HIDDEN = 128
VOCAB = 28
NUM_SEGMENTS = 10000

VOCAB_PAD = 32
NC = 2
NS = 16
NW = NC * NS
CHUNK = 128
N_CH = N_EDGES // CHUNK
CH_PER_W = N_CH // NW
N_TAIL = N_CH - CH_PER_W * NW
CH_ALL = CH_PER_W + 1
E_REG = CH_PER_W * CHUNK
E_PAD = CH_ALL * CHUNK
C_PAD = 10240
CYC_BLOCKS = C_PAD // 128
HIST_SP = VOCAB_PAD * C_PAD
PAD_SLOT = NUM_SEGMENTS
ZSLICE = HIST_SP // NS


def _sc_body(x_hbm, a2c_hbm, hist_hbm,
             x_v, ed_v, fidx_v, ones_v, zer_v, hist_s,
             sem_in, sem_sc):
    c = lax.axis_index("c")
    s = lax.axis_index("s")
    w = c * NS + s
    tail_ch = CH_PER_W * NW + (w & 3)

    cp_x = pltpu.async_copy(x_hbm, x_v, sem_in)
    cp_r = pltpu.async_copy(
        a2c_hbm.at[:, pl.ds(pl.multiple_of(w * E_REG, CHUNK), E_REG)],
        ed_v.at[:, pl.ds(0, E_REG)], sem_in)
    cp_t = pltpu.async_copy(
        a2c_hbm.at[:, pl.ds(pl.multiple_of(tail_ch * CHUNK, CHUNK), CHUNK)],
        ed_v.at[:, pl.ds(E_REG, CHUNK)], sem_in)

    zero16 = jnp.zeros((16,), jnp.float32)

    def zbody(i, carry):
        for k in range(8):
            zer_v[pl.ds(i * 128 + k * 16, 16)] = zero16
        return carry

    lax.fori_loop(0, ZSLICE // 128, zbody, 0)
    pltpu.sync_copy(zer_v, hist_s.at[pl.ds(s * ZSLICE, ZSLICE)])

    one16 = jnp.ones((16,), jnp.float32)
    for k in range(CHUNK // 16):
        ones_v[pl.ds(k * 16, 16)] = one16

    cp_x.wait()
    cp_r.wait()
    cp_t.wait()

    def chunk_fidx(j, mask_to_pad):
        for k in range(CHUNK // 16):
            s16 = ed_v[0, pl.ds(j * CHUNK + k * 16, 16)]
            d16 = ed_v[1, pl.ds(j * CHUNK + k * 16, 16)]
            t16 = plsc.load_gather(x_v, [s16])
            f16 = t16 * C_PAD + d16
            if mask_to_pad is not None:
                f16 = jnp.where(mask_to_pad, jnp.int32(PAD_SLOT), f16)
            fidx_v[j, pl.ds(k * 16, 16)] = f16
        pltpu.async_copy(ones_v, hist_s.at[fidx_v.at[j]], sem_sc, add=True)

    def cbody(j, carry):
        chunk_fidx(j, None)
        return carry

    lax.fori_loop(0, CH_PER_W, cbody, 0)
    chunk_fidx(CH_PER_W, w >= N_TAIL)

    def dbody(j, carry):
        pltpu.make_async_copy(ones_v, hist_s.at[fidx_v.at[0]], sem_sc).wait()
        return carry

    lax.fori_loop(0, CH_ALL, dbody, 0)

    plsc.subcore_barrier()

    pltpu.sync_copy(hist_s.at[pl.ds(s * ZSLICE, ZSLICE)],
                    zer_v.at[pl.ds(0, ZSLICE)])
    pltpu.sync_copy(zer_v.at[pl.ds(0, ZSLICE)],
                    hist_hbm.at[pl.ds(c * HIST_SP + s * ZSLICE, ZSLICE)])


_sc_hist = pl.kernel(
    _sc_body,
    out_type=jax.ShapeDtypeStruct((NC * HIST_SP,), jnp.float32),
    mesh=plsc.VectorSubcoreMesh(core_axis_name="c", subcore_axis_name="s"),
    compiler_params=pltpu.CompilerParams(needs_layout_passes=False),
    scratch_types=[
        pltpu.VMEM((N_ATOMS,), jnp.int32),
        pltpu.VMEM((2, E_PAD), jnp.int32),
        pltpu.VMEM((CH_ALL, CHUNK), jnp.int32),
        pltpu.VMEM((CHUNK,), jnp.float32),
        pltpu.VMEM((ZSLICE,), jnp.float32),
        pltpu.VMEM_SHARED((HIST_SP,), jnp.float32),
        pltpu.SemaphoreType.DMA,
        pltpu.SemaphoreType.DMA,
    ],
)


MM_GRID = 10
MM_BJ = CYC_BLOCKS // MM_GRID


def _mm_body(hist_ref, emb_ref, out_ref):
    e = emb_ref[...]
    for bj in range(MM_BJ):
        h = hist_ref[0, :, bj, :] + hist_ref[1, :, bj, :]
        out_ref[pl.ds(bj * 128, 128), :] = lax.dot_general(
            h, e, (((0,), (0,)), ((), ())),
            preferred_element_type=jnp.float32)


def _tc_expand(hist4, embp):
    return pl.pallas_call(
        _mm_body,
        grid=(MM_GRID,),
        in_specs=[
            pl.BlockSpec((NC, VOCAB_PAD, MM_BJ, 128), lambda i: (0, 0, i, 0)),
            pl.BlockSpec((VOCAB_PAD, HIDDEN), lambda i: (0, 0)),
        ],
        out_specs=pl.BlockSpec((MM_BJ * 128, HIDDEN), lambda i: (i, 0)),
        out_shape=jax.ShapeDtypeStruct((NUM_SEGMENTS, HIDDEN), jnp.float32),
    )(hist4, embp)


@jax.jit
def kernel(x, atom_to_cycle, emb_weight):
    hist = _sc_hist(x, atom_to_cycle)
    hist4 = hist.reshape(NC, VOCAB_PAD, CYC_BLOCKS, 128)
    embp = jnp.pad(emb_weight, ((0, VOCAB_PAD - VOCAB), (0, 0)))
    return _tc_expand(hist4, embp)


# speedup vs baseline: 49.9578x; 1.0674x over previous
import functools

import jax
import jax.numpy as jnp
from jax import lax
from jax.experimental import pallas as pl
from jax.experimental.pallas import tpu as pltpu
from jax.experimental.pallas import tpu_sc as plsc

N_ATOMS = 10000
N_EDGES = 320000
HIDDEN = 128
VOCAB = 28
NUM_SEGMENTS = 10000

VOCAB_PAD = 32
NC = 2
NS = 16
NW = NC * NS
CHUNK = 128
N_CH = N_EDGES // CHUNK
CH_PER_W = N_CH // NW
N_TAIL = N_CH - CH_PER_W * NW
CH_ALL = CH_PER_W + 1
E_REG = CH_PER_W * CHUNK
E_PAD = CH_ALL * CHUNK
C_PAD = 10240
CYC_BLOCKS = C_PAD // 128
HIST_SP = VOCAB_PAD * C_PAD
PAD_SLOT = NUM_SEGMENTS
ZSLICE = HIST_SP // NS


def _sc_body(x_hbm, a2c_hbm, hist_hbm,
             x_v, ed_v, fidx_v, ones_v, zer_v, hist_s,
             sem_in, sem_sc):
    c = lax.axis_index("c")
    s = lax.axis_index("s")
    w = c * NS + s
    tail_ch = CH_PER_W * NW + (w & 3)

    cp_x = pltpu.async_copy(x_hbm, x_v, sem_in)
    cp_r = pltpu.async_copy(
        a2c_hbm.at[:, pl.ds(pl.multiple_of(w * E_REG, CHUNK), E_REG)],
        ed_v.at[:, pl.ds(0, E_REG)], sem_in)
    cp_t = pltpu.async_copy(
        a2c_hbm.at[:, pl.ds(pl.multiple_of(tail_ch * CHUNK, CHUNK), CHUNK)],
        ed_v.at[:, pl.ds(E_REG, CHUNK)], sem_in)

    zero16 = jnp.zeros((16,), jnp.float32)

    def zbody(i, carry):
        for k in range(8):
            zer_v[pl.ds(i * 128 + k * 16, 16)] = zero16
        return carry

    lax.fori_loop(0, ZSLICE // 128, zbody, 0)
    pltpu.sync_copy(zer_v, hist_s.at[pl.ds(s * ZSLICE, ZSLICE)])

    one16 = jnp.ones((16,), jnp.float32)
    for k in range(CHUNK // 16):
        ones_v[pl.ds(k * 16, 16)] = one16

    cp_x.wait()
    cp_r.wait()
    cp_t.wait()

    def chunk_fidx(j, mask_to_pad):
        for k in range(CHUNK // 16):
            s16 = ed_v[0, pl.ds(j * CHUNK + k * 16, 16)]
            d16 = ed_v[1, pl.ds(j * CHUNK + k * 16, 16)]
            t16 = plsc.load_gather(x_v, [s16])
            f16 = t16 * C_PAD + d16
            if mask_to_pad is not None:
                f16 = jnp.where(mask_to_pad, jnp.int32(PAD_SLOT), f16)
            fidx_v[j, pl.ds(k * 16, 16)] = f16
        pltpu.async_copy(ones_v, hist_s.at[fidx_v.at[j]], sem_sc, add=True)

    def cbody(j, carry):
        chunk_fidx(j, None)
        return carry

    lax.fori_loop(0, CH_PER_W, cbody, 0)
    chunk_fidx(CH_PER_W, w >= N_TAIL)

    def dbody(j, carry):
        pltpu.make_async_copy(ones_v, hist_s.at[fidx_v.at[0]], sem_sc).wait()
        return carry

    lax.fori_loop(0, CH_ALL, dbody, 0)

    plsc.subcore_barrier()

    pltpu.sync_copy(hist_s.at[pl.ds(s * ZSLICE, ZSLICE)],
                    zer_v.at[pl.ds(0, ZSLICE)])
    pltpu.sync_copy(zer_v.at[pl.ds(0, ZSLICE)],
                    hist_hbm.at[pl.ds(c * HIST_SP + s * ZSLICE, ZSLICE)])


_sc_hist = pl.kernel(
    _sc_body,
    out_type=jax.ShapeDtypeStruct((NC * HIST_SP,), jnp.float32),
    mesh=plsc.VectorSubcoreMesh(core_axis_name="c", subcore_axis_name="s"),
    compiler_params=pltpu.CompilerParams(needs_layout_passes=False),
    scratch_types=[
        pltpu.VMEM((N_ATOMS,), jnp.int32),
        pltpu.VMEM((2, E_PAD), jnp.int32),
        pltpu.VMEM((CH_ALL, CHUNK), jnp.int32),
        pltpu.VMEM((CHUNK,), jnp.float32),
        pltpu.VMEM((ZSLICE,), jnp.float32),
        pltpu.VMEM_SHARED((HIST_SP,), jnp.float32),
        pltpu.SemaphoreType.DMA,
        pltpu.SemaphoreType.DMA,
    ],
)


MM_GRID = 5
MM_BJ = CYC_BLOCKS // MM_GRID


def _mm_body(hist_ref, emb_ref, out_ref):
    e = emb_ref[...]
    h = hist_ref[0] + hist_ref[1]
    hw = jnp.concatenate([h[:, bj, :] for bj in range(MM_BJ)], axis=1)
    out_ref[...] = lax.dot_general(
        hw, e, (((0,), (0,)), ((), ())),
        preferred_element_type=jnp.float32)


def _tc_expand(hist4, embp):
    return pl.pallas_call(
        _mm_body,
        grid=(MM_GRID,),
        in_specs=[
            pl.BlockSpec((NC, VOCAB_PAD, MM_BJ, 128), lambda i: (0, 0, i, 0)),
            pl.BlockSpec((VOCAB_PAD, HIDDEN), lambda i: (0, 0)),
        ],
        out_specs=pl.BlockSpec((MM_BJ * 128, HIDDEN), lambda i: (i, 0)),
        out_shape=jax.ShapeDtypeStruct((NUM_SEGMENTS, HIDDEN), jnp.float32),
    )(hist4, embp)


@jax.jit
def kernel(x, atom_to_cycle, emb_weight):
    hist = _sc_hist(x, atom_to_cycle)
    hist4 = hist.reshape(NC, VOCAB_PAD, CYC_BLOCKS, 128)
    embp = jnp.pad(emb_weight, ((0, VOCAB_PAD - VOCAB), (0, 0)))
    return _tc_expand(hist4, embp)


# speedup vs baseline: 50.5177x; 1.0112x over previous
import functools

import jax
import jax.numpy as jnp
from jax import lax
from jax.experimental import pallas as pl
from jax.experimental.pallas import tpu as pltpu
from jax.experimental.pallas import tpu_sc as plsc

N_ATOMS = 10000
N_EDGES = 320000
HIDDEN = 128
VOCAB = 28
NUM_SEGMENTS = 10000

NC = 2
NS = 16
NW = NC * NS
CHUNK = 128
N_CH = N_EDGES // CHUNK
CH_PER_W = N_CH // NW
N_TAIL = N_CH - CH_PER_W * NW
CH_ALL = CH_PER_W + 1
E_REG = CH_PER_W * CHUNK
E_PAD = CH_ALL * CHUNK
C_PAD = 10240
CYC_BLOCKS = C_PAD // 128
HIST_SP = VOCAB * C_PAD
PAD_SLOT = NUM_SEGMENTS
ZSLICE = HIST_SP // NS


def _sc_body(x_hbm, a2c_hbm, hist_hbm,
             x_v, ed_v, fidx_v, ones_v, zer_v, hist_s,
             sem_in, sem_sc):
    c = lax.axis_index("c")
    s = lax.axis_index("s")
    w = c * NS + s
    tail_ch = CH_PER_W * NW + (w & 3)

    cp_x = pltpu.async_copy(x_hbm, x_v, sem_in)
    cp_r = pltpu.async_copy(
        a2c_hbm.at[:, pl.ds(pl.multiple_of(w * E_REG, CHUNK), E_REG)],
        ed_v.at[:, pl.ds(0, E_REG)], sem_in)
    cp_t = pltpu.async_copy(
        a2c_hbm.at[:, pl.ds(pl.multiple_of(tail_ch * CHUNK, CHUNK), CHUNK)],
        ed_v.at[:, pl.ds(E_REG, CHUNK)], sem_in)

    zero16 = jnp.zeros((16,), jnp.float32)

    def zbody(i, carry):
        for k in range(8):
            zer_v[pl.ds(i * 128 + k * 16, 16)] = zero16
        return carry

    lax.fori_loop(0, ZSLICE // 128, zbody, 0)
    pltpu.sync_copy(zer_v, hist_s.at[pl.ds(s * ZSLICE, ZSLICE)])

    one16 = jnp.ones((16,), jnp.float32)
    for k in range(CHUNK // 16):
        ones_v[pl.ds(k * 16, 16)] = one16

    cp_x.wait()
    cp_r.wait()
    cp_t.wait()

    def chunk_fidx(j, mask_to_pad):
        for k in range(CHUNK // 16):
            s16 = ed_v[0, pl.ds(j * CHUNK + k * 16, 16)]
            d16 = ed_v[1, pl.ds(j * CHUNK + k * 16, 16)]
            t16 = plsc.load_gather(x_v, [s16])
            f16 = t16 * C_PAD + d16
            if mask_to_pad is not None:
                f16 = jnp.where(mask_to_pad, jnp.int32(PAD_SLOT), f16)
            fidx_v[j, pl.ds(k * 16, 16)] = f16
        pltpu.async_copy(ones_v, hist_s.at[fidx_v.at[j]], sem_sc, add=True)

    def cbody(j, carry):
        chunk_fidx(j, None)
        return carry

    lax.fori_loop(0, CH_PER_W, cbody, 0)
    chunk_fidx(CH_PER_W, w >= N_TAIL)

    def dbody(j, carry):
        pltpu.make_async_copy(ones_v, hist_s.at[fidx_v.at[0]], sem_sc).wait()
        return carry

    lax.fori_loop(0, CH_ALL, dbody, 0)

    plsc.subcore_barrier()

    pltpu.sync_copy(hist_s.at[pl.ds(s * ZSLICE, ZSLICE)],
                    zer_v.at[pl.ds(0, ZSLICE)])
    pltpu.sync_copy(zer_v.at[pl.ds(0, ZSLICE)],
                    hist_hbm.at[pl.ds(c * HIST_SP + s * ZSLICE, ZSLICE)])


_sc_hist = pl.kernel(
    _sc_body,
    out_type=jax.ShapeDtypeStruct((NC * HIST_SP,), jnp.float32),
    mesh=plsc.VectorSubcoreMesh(core_axis_name="c", subcore_axis_name="s"),
    compiler_params=pltpu.CompilerParams(needs_layout_passes=False),
    scratch_types=[
        pltpu.VMEM((N_ATOMS,), jnp.int32),
        pltpu.VMEM((2, E_PAD), jnp.int32),
        pltpu.VMEM((CH_ALL, CHUNK), jnp.int32),
        pltpu.VMEM((CHUNK,), jnp.float32),
        pltpu.VMEM((ZSLICE,), jnp.float32),
        pltpu.VMEM_SHARED((HIST_SP,), jnp.float32),
        pltpu.SemaphoreType.DMA,
        pltpu.SemaphoreType.DMA,
    ],
)


MM_GRID = 5
MM_BJ = CYC_BLOCKS // MM_GRID


def _mm_body(hist_ref, emb_ref, out_ref):
    e = emb_ref[...]
    h = hist_ref[0] + hist_ref[1]
    hw = jnp.concatenate([h[:, bj, :] for bj in range(MM_BJ)], axis=1)
    out_ref[...] = lax.dot_general(
        hw, e, (((0,), (0,)), ((), ())),
        preferred_element_type=jnp.float32)


def _tc_expand(hist4, embp):
    return pl.pallas_call(
        _mm_body,
        grid=(MM_GRID,),
        in_specs=[
            pl.BlockSpec((NC, VOCAB, MM_BJ, 128), lambda i: (0, 0, i, 0)),
            pl.BlockSpec((VOCAB, HIDDEN), lambda i: (0, 0)),
        ],
        out_specs=pl.BlockSpec((MM_BJ * 128, HIDDEN), lambda i: (i, 0)),
        out_shape=jax.ShapeDtypeStruct((NUM_SEGMENTS, HIDDEN), jnp.float32),
    )(hist4, embp)


@jax.jit
def kernel(x, atom_to_cycle, emb_weight):
    hist = _sc_hist(x, atom_to_cycle)
    hist4 = hist.reshape(NC, VOCAB, CYC_BLOCKS, 128)
    return _tc_expand(hist4, emb_weight)
